# baseline (speedup 1.0000x reference)
import functools

import jax
import jax.numpy as jnp
from jax import lax
from jax.experimental import pallas as pl
from jax.experimental.pallas import tpu as pltpu
from jax.experimental.pallas import tpu_sc as plsc

N = 10000
E = 320000
D = 128

NC = 2
NS = 16
NW = NC * NS
EPT = E // NW
NPAD = 10240
HSIZE = 2 * NPAD
HALF = NPAD // 2
CHUNK = 125
EPS = E // NS
NCHUNK = EPS // CHUNK
AGG_PER_TILE = HALF // NS
ZROWS = 64

_mesh = plsc.VectorSubcoreMesh(
    core_axis_name="c", subcore_axis_name="s", num_cores=NC, num_subcores=NS)

_sc_params = pltpu.CompilerParams(needs_layout_passes=False)


def _wid():
    return lax.axis_index("s") * NC + lax.axis_index("c")


@functools.partial(
    pl.kernel,
    out_type=(
        jax.ShapeDtypeStruct((NW * HSIZE,), jnp.float32),
        jax.ShapeDtypeStruct((3 * E,), jnp.int32),
    ),
    mesh=_mesh,
    compiler_params=_sc_params,
    scratch_types=[
        pltpu.VMEM((EPT,), jnp.int32),
        pltpu.VMEM((EPT,), jnp.int32),
        pltpu.VMEM((HSIZE,), jnp.float32),
        pltpu.VMEM((EPT,), jnp.int32),
        pltpu.VMEM((EPT,), jnp.int32),
    ],
)
def _sc_degrees(src_hbm, dst_hbm, out_hbm, ldx_hbm, sidx_v, didx_v, hist_v,
                l0_v, l1_v):
    w = _wid()

    pltpu.sync_copy(src_hbm.at[pl.ds(w * EPT, EPT)], sidx_v)
    pltpu.sync_copy(dst_hbm.at[pl.ds(w * EPT, EPT)], didx_v)

    zero16 = jnp.zeros((16,), dtype=jnp.float32)
    one16 = jnp.full((16,), 1.0, dtype=jnp.float32)

    def fill_zero(i, _):
        hist_v[pl.ds(i * 16, 16)] = zero16
        return ()
    lax.fori_loop(0, HSIZE // 16, fill_zero, (), unroll=8)

    def scatter(j, _):
        sl = pl.ds(j * 16, 16)
        sv = sidx_v[sl]
        plsc.addupdate_scatter(hist_v, [sv], one16)
        dv = didx_v[sl]
        plsc.addupdate_scatter(hist_v, [dv + NPAD], one16)
        l0_v[sl] = jnp.where(dv < HALF, dv, HALF)
        l1_v[sl] = jnp.where(dv >= HALF, dv - HALF, HALF)
        return ()
    lax.fori_loop(0, EPT // 16, scatter, (), unroll=4)

    pltpu.sync_copy(hist_v, out_hbm.at[pl.ds(w * HSIZE, HSIZE)])
    pltpu.sync_copy(sidx_v, ldx_hbm.at[pl.ds(w * EPT, EPT)])
    pltpu.sync_copy(l0_v, ldx_hbm.at[pl.ds(E + w * EPT, EPT)])
    pltpu.sync_copy(l1_v, ldx_hbm.at[pl.ds(2 * E + w * EPT, EPT)])


@functools.partial(
    pl.kernel,
    out_type=jax.ShapeDtypeStruct((NC, HALF, D), jnp.float32),
    mesh=_mesh,
    compiler_params=_sc_params,
    scratch_types=[
        pltpu.VMEM((NCHUNK, CHUNK), jnp.int32),
        pltpu.VMEM((NCHUNK, CHUNK), jnp.int32),
        [pltpu.VMEM((CHUNK, D), jnp.float32)] * 2,
        pltpu.VMEM((ZROWS, D), jnp.float32),
        pltpu.VMEM_SHARED((HALF + 8, D), jnp.float32),
        [pltpu.SemaphoreType.DMA] * 2,
    ],
)
def _sc_aggregate(h_hbm, idx_hbm, out_hbm, sidx_v, ldx_v, rows, z_v,
                  agg_sh, gsem):
    c = lax.axis_index("c")
    s = lax.axis_index("s")

    pltpu.sync_copy(idx_hbm.at[0, s], sidx_v)
    pltpu.sync_copy(idx_hbm.at[1 + c, s], ldx_v)

    zero16 = jnp.zeros((16,), dtype=jnp.float32)

    def fill_zero(i, _):
        for cc in range(D // 16):
            z_v[i, pl.ds(cc * 16, 16)] = zero16
        return ()
    lax.fori_loop(0, ZROWS, fill_zero, ())

    row0 = s * AGG_PER_TILE

    def zero_chunk(k, _):
        pltpu.sync_copy(z_v, agg_sh.at[pl.ds(row0 + k * ZROWS, ZROWS)])
        return ()
    lax.fori_loop(0, AGG_PER_TILE // ZROWS, zero_chunk, ())

    @pl.when(s == 0)
    def _():
        pltpu.sync_copy(z_v.at[pl.ds(0, 8)], agg_sh.at[pl.ds(HALF, 8)])

    plsc.subcore_barrier()

    def _gather(j, b):
        pltpu.async_copy(h_hbm.at[sidx_v.at[j]], rows[b], gsem[b])

    def _gather_wait(j, b):
        pltpu.make_async_copy(h_hbm.at[sidx_v.at[j]], rows[b], gsem[b]).wait()

    _gather(0, 0)

    def edge_duo(t, _):
        j0 = 2 * t
        _gather(j0 + 1, 1)
        _gather_wait(j0, 0)
        pltpu.sync_copy(rows[0], agg_sh.at[ldx_v.at[j0]], add=True)
        _gather(jnp.minimum(j0 + 2, NCHUNK - 1), 0)
        _gather_wait(j0 + 1, 1)
        pltpu.sync_copy(rows[1], agg_sh.at[ldx_v.at[j0 + 1]], add=True)
        return ()
    lax.fori_loop(0, NCHUNK // 2, edge_duo, ())
    _gather_wait(NCHUNK - 1, 0)
    plsc.subcore_barrier()

    pltpu.sync_copy(agg_sh.at[pl.ds(row0, AGG_PER_TILE)],
                    out_hbm.at[c, pl.ds(row0, AGG_PER_TILE)])


BLK = 512
GRID = NPAD // BLK


def _scale_matmul_body(x_ref, degs_ref, w_ref, o_ref):
    deg = jnp.sum(degs_ref[...], axis=0)
    scale = lax.rsqrt(jnp.maximum(deg, 1.0))
    o_ref[...] = jnp.dot(x_ref[...] * scale, w_ref[...],
                         preferred_element_type=jnp.float32)


def _tc_scale_matmul(feats, degs_col, w):
    return pl.pallas_call(
        _scale_matmul_body,
        grid=(GRID,),
        in_specs=[
            pl.BlockSpec((BLK, D), lambda i: (i, 0)),
            pl.BlockSpec((NW, BLK, 1), lambda i: (0, i, 0)),
            pl.BlockSpec((D, D), lambda i: (0, 0)),
        ],
        out_specs=pl.BlockSpec((BLK, D), lambda i: (i, 0)),
        out_shape=jax.ShapeDtypeStruct((N, D), jnp.float32),
    )(feats, degs_col, w)


def _finalize_body(agg_ref, degd_ref, b_ref, o_ref):
    deg = jnp.sum(degd_ref[...], axis=0)
    scale = lax.rsqrt(jnp.maximum(deg, 1.0))
    o_ref[...] = agg_ref[...] * scale + b_ref[...]


def _tc_finalize(agg, degd_col, b2col):
    return pl.pallas_call(
        _finalize_body,
        grid=(GRID,),
        in_specs=[
            pl.BlockSpec((BLK, D), lambda i: (i, 0)),
            pl.BlockSpec((NW, BLK, 1), lambda i: (0, i, 0)),
            pl.BlockSpec((1, D), lambda i: (0, 0)),
        ],
        out_specs=pl.BlockSpec((BLK, D), lambda i: (i, 0)),
        out_shape=jax.ShapeDtypeStruct((N, D), jnp.float32),
    )(agg, degd_col, b2col)


def kernel(feats, edge_index, W1, b1, W2, b2):
    del W1, b1
    ei = edge_index.astype(jnp.int32)
    src_flat = ei[0]
    dst_flat = ei[1]

    deg, idx = _sc_degrees(src_flat, dst_flat)
    idx4 = idx.reshape(3, NS, NCHUNK, CHUNK)
    deg = deg.reshape(NW, 2, NPAD)
    degs_col = deg[:, 0, :, None]
    degd_col = deg[:, 1, :, None]
    h = _tc_scale_matmul(feats, degs_col, W2)
    agg = _sc_aggregate(h, idx4)
    return _tc_finalize(agg.reshape(NPAD, D), degd_col, b2.reshape(1, D))


# speedup vs baseline: 1.0243x; 1.0243x over previous
import functools

import jax
import jax.numpy as jnp
from jax import lax
from jax.experimental import pallas as pl
from jax.experimental.pallas import tpu as pltpu
from jax.experimental.pallas import tpu_sc as plsc

N = 10000
E = 320000
D = 128

NC = 2
NS = 16
NW = NC * NS
EPT = E // NW
NPAD = 10240
HSIZE = 2 * NPAD
HALF = NPAD // 2
CHUNK = 125
EPS = E // NS
NCHUNK = EPS // CHUNK
AGG_PER_TILE = HALF // NS
ZROWS = 64

_mesh = plsc.VectorSubcoreMesh(
    core_axis_name="c", subcore_axis_name="s", num_cores=NC, num_subcores=NS)

_sc_params = pltpu.CompilerParams(needs_layout_passes=False)


def _wid():
    return lax.axis_index("s") * NC + lax.axis_index("c")


@functools.partial(
    pl.kernel,
    out_type=jax.ShapeDtypeStruct((NW * HSIZE,), jnp.float32),
    mesh=_mesh,
    compiler_params=_sc_params,
    scratch_types=[
        pltpu.VMEM((EPT,), jnp.int32),
        pltpu.VMEM((EPT,), jnp.int32),
        pltpu.VMEM((HSIZE,), jnp.float32),
    ],
)
def _sc_degrees(src_hbm, dst_hbm, out_hbm, sidx_v, didx_v, hist_v):
    w = _wid()

    pltpu.sync_copy(src_hbm.at[pl.ds(w * EPT, EPT)], sidx_v)
    pltpu.sync_copy(dst_hbm.at[pl.ds(w * EPT, EPT)], didx_v)

    zero16 = jnp.zeros((16,), dtype=jnp.float32)
    one16 = jnp.full((16,), 1.0, dtype=jnp.float32)

    def fill_zero(i, _):
        hist_v[pl.ds(i * 16, 16)] = zero16
        return ()
    lax.fori_loop(0, HSIZE // 16, fill_zero, (), unroll=8)

    def scatter(j, _):
        sl = pl.ds(j * 16, 16)
        sv = sidx_v[sl]
        plsc.addupdate_scatter(hist_v, [sv], one16)
        dv = didx_v[sl]
        plsc.addupdate_scatter(hist_v, [dv + NPAD], one16)
        return ()
    lax.fori_loop(0, EPT // 16, scatter, (), unroll=4)

    pltpu.sync_copy(hist_v, out_hbm.at[pl.ds(w * HSIZE, HSIZE)])


@functools.partial(
    pl.kernel,
    out_type=jax.ShapeDtypeStruct((NC, HALF, D), jnp.float32),
    mesh=_mesh,
    compiler_params=_sc_params,
    scratch_types=[
        pltpu.VMEM((NCHUNK, CHUNK), jnp.int32),
        pltpu.VMEM((NCHUNK, CHUNK), jnp.int32),
        [pltpu.VMEM((CHUNK, D), jnp.float32)] * 2,
        pltpu.VMEM((ZROWS, D), jnp.float32),
        pltpu.VMEM_SHARED((HALF + 8, D), jnp.float32),
        [pltpu.SemaphoreType.DMA] * 2,
    ],
)
def _sc_aggregate(h_hbm, src_hbm, ldx_hbm, out_hbm, sidx_v, ldx_v, rows, z_v,
                  agg_sh, gsem):
    c = lax.axis_index("c")
    s = lax.axis_index("s")

    pltpu.sync_copy(src_hbm.at[s], sidx_v)
    pltpu.sync_copy(ldx_hbm.at[c, s], ldx_v)

    zero16 = jnp.zeros((16,), dtype=jnp.float32)

    def fill_zero(i, _):
        for cc in range(D // 16):
            z_v[i, pl.ds(cc * 16, 16)] = zero16
        return ()
    lax.fori_loop(0, ZROWS, fill_zero, ())

    row0 = s * AGG_PER_TILE

    def zero_chunk(k, _):
        pltpu.sync_copy(z_v, agg_sh.at[pl.ds(row0 + k * ZROWS, ZROWS)])
        return ()
    lax.fori_loop(0, AGG_PER_TILE // ZROWS, zero_chunk, ())

    @pl.when(s == 0)
    def _():
        pltpu.sync_copy(z_v.at[pl.ds(0, 8)], agg_sh.at[pl.ds(HALF, 8)])

    plsc.subcore_barrier()

    def _gather(j, b):
        pltpu.async_copy(h_hbm.at[sidx_v.at[j]], rows[b], gsem[b])

    def _gather_wait(j, b):
        pltpu.make_async_copy(h_hbm.at[sidx_v.at[j]], rows[b], gsem[b]).wait()

    _gather(0, 0)

    def edge_duo(t, _):
        j0 = 2 * t
        _gather(j0 + 1, 1)
        _gather_wait(j0, 0)
        pltpu.sync_copy(rows[0], agg_sh.at[ldx_v.at[j0]], add=True)
        _gather(jnp.minimum(j0 + 2, NCHUNK - 1), 0)
        _gather_wait(j0 + 1, 1)
        pltpu.sync_copy(rows[1], agg_sh.at[ldx_v.at[j0 + 1]], add=True)
        return ()
    lax.fori_loop(0, NCHUNK // 2, edge_duo, ())
    _gather_wait(NCHUNK - 1, 0)
    plsc.subcore_barrier()

    pltpu.sync_copy(agg_sh.at[pl.ds(row0, AGG_PER_TILE)],
                    out_hbm.at[c, pl.ds(row0, AGG_PER_TILE)])


BLK = 512
GRID = NPAD // BLK


def _scale_matmul_body(x_ref, degs_ref, w_ref, o_ref):
    deg = jnp.sum(degs_ref[...], axis=0)
    scale = lax.rsqrt(jnp.maximum(deg, 1.0))
    o_ref[...] = jnp.dot(x_ref[...] * scale, w_ref[...],
                         preferred_element_type=jnp.float32)


def _tc_scale_matmul(feats, degs_col, w):
    return pl.pallas_call(
        _scale_matmul_body,
        grid=(GRID,),
        in_specs=[
            pl.BlockSpec((BLK, D), lambda i: (i, 0)),
            pl.BlockSpec((NW, BLK, 1), lambda i: (0, i, 0)),
            pl.BlockSpec((D, D), lambda i: (0, 0)),
        ],
        out_specs=pl.BlockSpec((BLK, D), lambda i: (i, 0)),
        out_shape=jax.ShapeDtypeStruct((N, D), jnp.float32),
    )(feats, degs_col, w)


def _finalize_body(agg_ref, degd_ref, b_ref, o_ref):
    deg = jnp.sum(degd_ref[...], axis=0)
    scale = lax.rsqrt(jnp.maximum(deg, 1.0))
    o_ref[...] = agg_ref[...] * scale + b_ref[...]


def _tc_finalize(agg, degd_col, b2col):
    return pl.pallas_call(
        _finalize_body,
        grid=(GRID,),
        in_specs=[
            pl.BlockSpec((BLK, D), lambda i: (i, 0)),
            pl.BlockSpec((NW, BLK, 1), lambda i: (0, i, 0)),
            pl.BlockSpec((1, D), lambda i: (0, 0)),
        ],
        out_specs=pl.BlockSpec((BLK, D), lambda i: (i, 0)),
        out_shape=jax.ShapeDtypeStruct((N, D), jnp.float32),
    )(agg, degd_col, b2col)


def kernel(feats, edge_index, W1, b1, W2, b2):
    del W1, b1
    ei = edge_index.astype(jnp.int32)
    src_flat = ei[0]
    dst_flat = ei[1]

    src3 = src_flat.reshape(NS, NCHUNK, CHUNK)
    dst3 = dst_flat.reshape(1, NS, NCHUNK, CHUNK)
    base = jnp.arange(NC, dtype=jnp.int32)[:, None, None, None] * HALF
    loc = dst3 - base
    ldx = jnp.where((loc >= 0) & (loc < HALF), loc, HALF)

    deg = _sc_degrees(src_flat, dst_flat)
    deg = deg.reshape(NW, 2, NPAD)
    degs_col = deg[:, 0, :, None]
    degd_col = deg[:, 1, :, None]
    h = _tc_scale_matmul(feats, degs_col, W2)
    agg = _sc_aggregate(h, src3, ldx)
    return _tc_finalize(agg.reshape(NPAD, D), degd_col, b2.reshape(1, D))


# speedup vs baseline: 1.1230x; 1.0963x over previous
import functools

import jax
import jax.numpy as jnp
from jax import lax
from jax.experimental import pallas as pl
from jax.experimental.pallas import tpu as pltpu
from jax.experimental.pallas import tpu_sc as plsc

N = 10000
E = 320000
D = 128

NC = 2
NS = 16
NW = NC * NS
EPT = E // NW
NPAD = 10240
HSIZE = 2 * NPAD
HALF = NPAD // 2
CHUNK = 125
EPS = E // NS
NCHUNK = EPS // CHUNK
AGG_PER_TILE = HALF // NS
ZROWS = 64

_mesh = plsc.VectorSubcoreMesh(
    core_axis_name="c", subcore_axis_name="s", num_cores=NC, num_subcores=NS)

_sc_params = pltpu.CompilerParams(needs_layout_passes=False)


def _wid():
    return lax.axis_index("s") * NC + lax.axis_index("c")


@functools.partial(
    pl.kernel,
    out_type=jax.ShapeDtypeStruct((NW * HSIZE,), jnp.float32),
    mesh=_mesh,
    compiler_params=_sc_params,
    scratch_types=[
        pltpu.VMEM((EPT,), jnp.int32),
        pltpu.VMEM((EPT,), jnp.int32),
        pltpu.VMEM((HSIZE,), jnp.float32),
    ],
)
def _sc_degrees(src_hbm, dst_hbm, out_hbm, sidx_v, didx_v, hist_v):
    w = _wid()

    pltpu.sync_copy(src_hbm.at[pl.ds(w * EPT, EPT)], sidx_v)
    pltpu.sync_copy(dst_hbm.at[pl.ds(w * EPT, EPT)], didx_v)

    zero16 = jnp.zeros((16,), dtype=jnp.float32)
    one16 = jnp.full((16,), 1.0, dtype=jnp.float32)

    def fill_zero(i, _):
        hist_v[pl.ds(i * 16, 16)] = zero16
        return ()
    lax.fori_loop(0, HSIZE // 16, fill_zero, (), unroll=8)

    def scatter(j, _):
        sl = pl.ds(j * 16, 16)
        sv = sidx_v[sl]
        plsc.addupdate_scatter(hist_v, [sv], one16)
        dv = didx_v[sl]
        plsc.addupdate_scatter(hist_v, [dv + NPAD], one16)
        return ()
    lax.fori_loop(0, EPT // 16, scatter, (), unroll=4)

    pltpu.sync_copy(hist_v, out_hbm.at[pl.ds(w * HSIZE, HSIZE)])


@functools.partial(
    pl.kernel,
    out_type=jax.ShapeDtypeStruct((NC, HALF, D), jnp.float32),
    mesh=_mesh,
    compiler_params=_sc_params,
    scratch_types=[
        pltpu.VMEM((NCHUNK, CHUNK), jnp.int32),
        pltpu.VMEM((NCHUNK, CHUNK), jnp.int32),
        [pltpu.VMEM((CHUNK, D), jnp.float32)] * 2,
        pltpu.VMEM((ZROWS, D), jnp.float32),
        pltpu.VMEM_SHARED((HALF + 8, D), jnp.float32),
        [pltpu.SemaphoreType.DMA] * 2,
    ],
)
def _sc_aggregate(h_hbm, src_hbm, ldx_hbm, out_hbm, sidx_v, ldx_v, rows, z_v,
                  agg_sh, gsem):
    c = lax.axis_index("c")
    s = lax.axis_index("s")

    pltpu.sync_copy(src_hbm.at[s], sidx_v)
    pltpu.sync_copy(ldx_hbm.at[c, s], ldx_v)

    zero16 = jnp.zeros((16,), dtype=jnp.float32)

    def fill_zero(i, _):
        for cc in range(D // 16):
            z_v[i, pl.ds(cc * 16, 16)] = zero16
        return ()
    lax.fori_loop(0, ZROWS, fill_zero, ())

    row0 = s * AGG_PER_TILE

    def zero_chunk(k, _):
        pltpu.sync_copy(z_v, agg_sh.at[pl.ds(row0 + k * ZROWS, ZROWS)])
        return ()
    lax.fori_loop(0, AGG_PER_TILE // ZROWS, zero_chunk, ())

    @pl.when(s == 0)
    def _():
        pltpu.sync_copy(z_v.at[pl.ds(0, 8)], agg_sh.at[pl.ds(HALF, 8)])

    plsc.subcore_barrier()

    def _gather(j, b):
        pltpu.async_copy(h_hbm.at[sidx_v.at[j]], rows[b], gsem[b])

    def _gather_wait(j, b):
        pltpu.make_async_copy(h_hbm.at[sidx_v.at[j]], rows[b], gsem[b]).wait()

    _gather(0, 0)

    def edge_duo(t, _):
        j0 = 2 * t
        _gather(j0 + 1, 1)
        _gather_wait(j0, 0)
        pltpu.sync_copy(rows[0], agg_sh.at[ldx_v.at[j0]], add=True)
        _gather(jnp.minimum(j0 + 2, NCHUNK - 1), 0)
        _gather_wait(j0 + 1, 1)
        pltpu.sync_copy(rows[1], agg_sh.at[ldx_v.at[j0 + 1]], add=True)
        return ()
    lax.fori_loop(0, NCHUNK // 2, edge_duo, ())
    _gather_wait(NCHUNK - 1, 0)
    plsc.subcore_barrier()

    pltpu.sync_copy(agg_sh.at[pl.ds(row0, AGG_PER_TILE)],
                    out_hbm.at[c, pl.ds(row0, AGG_PER_TILE)])


BLK = 512
GRID = NPAD // BLK


def _scale_matmul_body(x_ref, degs_ref, w_ref, o_ref):
    deg = jnp.sum(degs_ref[...], axis=0)
    scale = lax.rsqrt(jnp.maximum(deg, 1.0))
    o_ref[...] = jnp.dot(x_ref[...] * scale, w_ref[...],
                         preferred_element_type=jnp.float32)


def _tc_scale_matmul(feats, degs_col, w):
    return pl.pallas_call(
        _scale_matmul_body,
        grid=(GRID,),
        in_specs=[
            pl.BlockSpec((BLK, D), lambda i: (i, 0)),
            pl.BlockSpec((NW, BLK, 1), lambda i: (0, i, 0)),
            pl.BlockSpec((D, D), lambda i: (0, 0)),
        ],
        out_specs=pl.BlockSpec((BLK, D), lambda i: (i, 0)),
        out_shape=jax.ShapeDtypeStruct((N, D), jnp.float32),
    )(feats, degs_col, w)


def _finalize_body(agg_ref, degd_ref, b_ref, o_ref):
    deg = jnp.sum(degd_ref[...], axis=0)
    scale = lax.rsqrt(jnp.maximum(deg, 1.0))
    o_ref[...] = agg_ref[...] * scale + b_ref[...]


def _tc_finalize(agg, degd_col, b2col):
    return pl.pallas_call(
        _finalize_body,
        grid=(GRID,),
        in_specs=[
            pl.BlockSpec((BLK, D), lambda i: (i, 0)),
            pl.BlockSpec((NW, BLK, 1), lambda i: (0, i, 0)),
            pl.BlockSpec((1, D), lambda i: (0, 0)),
        ],
        out_specs=pl.BlockSpec((BLK, D), lambda i: (i, 0)),
        out_shape=jax.ShapeDtypeStruct((N, D), jnp.float32),
    )(agg, degd_col, b2col)


def kernel(feats, edge_index, W1, b1, W2, b2):
    del W1, b1
    ei = edge_index.astype(jnp.int32)
    src_flat = ei[0]
    dst_flat = ei[1]

    src3 = src_flat.reshape(NS, NCHUNK, CHUNK)
    dst3 = dst_flat.reshape(1, NS, NCHUNK, CHUNK)
    base = jnp.arange(NC, dtype=jnp.int32)[:, None, None, None] * HALF
    loc = dst3 - base
    trash = HALF + (jnp.arange(CHUNK, dtype=jnp.int32) % 8)
    ldx = jnp.where((loc >= 0) & (loc < HALF), loc, trash)

    deg = _sc_degrees(src_flat, dst_flat)
    deg = deg.reshape(NW, 2, NPAD)
    degs_col = deg[:, 0, :, None]
    degd_col = deg[:, 1, :, None]
    h = _tc_scale_matmul(feats, degs_col, W2)
    agg = _sc_aggregate(h, src3, ldx)
    return _tc_finalize(agg.reshape(NPAD, D), degd_col, b2.reshape(1, D))
